# baseline (speedup 1.0000x reference)
import functools

import jax
import jax.numpy as jnp
from jax import lax
from jax.experimental import pallas as pl
from jax.experimental.pallas import tpu as pltpu
from jax.experimental.pallas import tpu_sc as plsc

N = 10000
E = 320000
G = 100
NC, NS, L = 2, 16, 16
NW = NC * NS
R = 320
NPAD = NW * R
C = 2000
K = 32
GP = 104
NEG_INF = float("-inf")


def _tc_ab_body(x_ref, wa_ref, wb_ref, b_ref, a_ref, bo_ref):
    xb = x_ref[...]
    wb = wb_ref[...]
    wd = wa_ref[...] - wb
    a_ref[...] = (
        jnp.dot(xb, wd, preferred_element_type=jnp.float32) + b_ref[...]
    )
    bo_ref[...] = jnp.dot(xb, wb, preferred_element_type=jnp.float32)


def _tc_ab(x, wa, wb, b, blk_rows):
    nrows, kdim = x.shape
    dout = wa.shape[1]
    grid = (nrows // blk_rows,)
    return pl.pallas_call(
        _tc_ab_body,
        grid=grid,
        in_specs=[
            pl.BlockSpec((blk_rows, kdim), lambda i: (i, 0)),
            pl.BlockSpec((kdim, dout), lambda i: (0, 0)),
            pl.BlockSpec((kdim, dout), lambda i: (0, 0)),
            pl.BlockSpec((1, dout), lambda i: (0, 0)),
        ],
        out_specs=[
            pl.BlockSpec((blk_rows, dout), lambda i: (i, 0)),
            pl.BlockSpec((blk_rows, dout), lambda i: (i, 0)),
        ],
        out_shape=[
            jax.ShapeDtypeStruct((nrows, dout), jnp.float32),
            jax.ShapeDtypeStruct((nrows, dout), jnp.float32),
        ],
    )(x, wa, wb, b)


def _tc_post_body(p_ref, wo1_ref, bo1_ref, wo2_ref, bo2_ref, o_ref):
    g = jnp.max(p_ref[...], axis=0)[:G]
    g = jnp.where(jnp.isfinite(g), g, 0.0)
    t = jax.nn.sigmoid(
        jnp.dot(g, wo1_ref[...], preferred_element_type=jnp.float32)
        + bo1_ref[...]
    )
    o_ref[...] = (
        jnp.dot(t, wo2_ref[...], preferred_element_type=jnp.float32)
        + bo2_ref[...]
    )


def _tc_post(p, wo1, bo1, wo2, bo2):
    return pl.pallas_call(
        _tc_post_body,
        out_shape=jax.ShapeDtypeStruct((G, wo2.shape[1]), jnp.float32),
    )(p, wo1, bo1, wo2, bo2)


def _worker_id():
    return lax.axis_index("s") * NC + lax.axis_index("c")


def _init_neg_inf(ref, nrows, d):
    ninf = jnp.full((L,), NEG_INF, jnp.float32)

    def body(r, _):
        for f in range(d // L):
            ref[r, pl.ds(f * L, L)] = ninf
        return 0

    lax.fori_loop(0, nrows, body, 0)


def _segmax_chunks(lo, b_hbm, src_hbm, dst_hbm, acc, dstv, srcv, mbuf,
                   gidx, gbuf, sem, d):
    def chunk_body(c, _):
        off = c * C
        pltpu.sync_copy(dst_hbm.at[pl.ds(off, C)], dstv)
        pltpu.sync_copy(src_hbm.at[pl.ds(off, C)], srcv)

        def filt(g, carry):
            gb = g * L
            vd = dstv[pl.ds(gb, L)]
            vs = srcv[pl.ds(gb, L)]
            m = (vd >= lo) & (vd < lo + R)
            mi = m.astype(jnp.int32)
            pos = carry + plsc.cumsum(mi) - mi
            packed = vs | ((vd - lo) << 14)
            plsc.store_scatter(mbuf, [pos], packed, mask=m)
            return carry + plsc.all_reduce_population_count(m)

        kvec = lax.fori_loop(0, C // L, filt, jnp.zeros((L,), jnp.int32))
        dummy = jnp.full((L,), R << 14, jnp.int32)
        iota = lax.iota(jnp.int32, L)
        plsc.store_scatter(mbuf, [kvec + iota], dummy)
        plsc.store_scatter(mbuf, [kvec + iota + L], dummy)
        kk = jnp.max(kvec)
        nb = (kk + K - 1) // K

        def sub(s, _):
            sb = s * K
            for g in range(K // L):
                pk = mbuf[pl.ds(sb + g * L, L)]
                gidx[pl.ds(g * L, L)] = pk & 0x3FFF
            pltpu.async_copy(b_hbm.at[gidx], gbuf, sem).wait()
            for g in range(K // L):
                dlv = mbuf[pl.ds(sb + g * L, L)] >> 14
                for e in range(L):
                    r = dlv[e]
                    for f in range(d // L):
                        sl = pl.ds(f * L, L)
                        acc[r, sl] = jnp.maximum(acc[r, sl],
                                                 gbuf[g * L + e, sl])
            return 0

        lax.fori_loop(0, nb, sub, 0)
        return 0

    lax.fori_loop(0, E // C, chunk_body, 0)


def _sigmoid_rows(lo, a_hbm, acc, gbuf, d):
    def hrow(rc, _):
        pltpu.sync_copy(a_hbm.at[pl.ds(lo + rc * K, K)], gbuf)
        for e in range(K):
            for f in range(d // L):
                sl = pl.ds(f * L, L)
                z = acc[rc * K + e, sl] + gbuf[e, sl]
                acc[rc * K + e, sl] = 1.0 / (1.0 + jnp.exp(-z))
        return 0

    lax.fori_loop(0, R // K, hrow, 0)


def _sc_layer1(a1, b1, src, dst):
    d = a1.shape[1]
    mesh = plsc.VectorSubcoreMesh(
        core_axis_name="c", subcore_axis_name="s", num_cores=NC,
        num_subcores=NS)

    @functools.partial(
        pl.kernel,
        out_type=jax.ShapeDtypeStruct((NPAD, d), jnp.float32),
        mesh=mesh,
        compiler_params=pltpu.CompilerParams(needs_layout_passes=False),
        scratch_types=[
            pltpu.VMEM((R + 1, d), jnp.float32),
            pltpu.VMEM((C,), jnp.int32),
            pltpu.VMEM((C,), jnp.int32),
            pltpu.VMEM((C + K,), jnp.int32),
            pltpu.VMEM((K,), jnp.int32),
            pltpu.VMEM((K, d), jnp.float32),
            pltpu.SemaphoreType.DMA,
        ],
    )
    def k(a_hbm, b_hbm, src_hbm, dst_hbm, h_hbm,
          acc, dstv, srcv, mbuf, gidx, gbuf, sem):
        wid = _worker_id()
        lo = wid * R
        _init_neg_inf(acc, R + 1, d)
        _segmax_chunks(lo, b_hbm, src_hbm, dst_hbm, acc, dstv, srcv,
                       mbuf, gidx, gbuf, sem, d)
        _sigmoid_rows(lo, a_hbm, acc, gbuf, d)
        pltpu.sync_copy(acc.at[pl.ds(0, R)], h_hbm.at[pl.ds(lo, R)])

    return k(a1, b1, src, dst)


def _sc_layer2(a2, b2, src, dst, batch_pad):
    d = a2.shape[1]
    mesh = plsc.VectorSubcoreMesh(
        core_axis_name="c", subcore_axis_name="s", num_cores=NC,
        num_subcores=NS)

    @functools.partial(
        pl.kernel,
        out_type=jax.ShapeDtypeStruct((NW, GP, d), jnp.float32),
        mesh=mesh,
        compiler_params=pltpu.CompilerParams(needs_layout_passes=False),
        scratch_types=[
            pltpu.VMEM((R + 1, d), jnp.float32),
            pltpu.VMEM((GP, d), jnp.float32),
            pltpu.VMEM((C,), jnp.int32),
            pltpu.VMEM((C,), jnp.int32),
            pltpu.VMEM((C + K,), jnp.int32),
            pltpu.VMEM((K,), jnp.int32),
            pltpu.VMEM((K, d), jnp.float32),
            pltpu.VMEM((R,), jnp.int32),
            pltpu.SemaphoreType.DMA,
        ],
    )
    def k(a_hbm, b_hbm, src_hbm, dst_hbm, batch_hbm, p_hbm,
          acc, pool, dstv, srcv, mbuf, gidx, gbuf, bbuf, sem):
        wid = _worker_id()
        lo = wid * R
        _init_neg_inf(acc, R + 1, d)
        _init_neg_inf(pool, GP, d)
        _segmax_chunks(lo, b_hbm, src_hbm, dst_hbm, acc, dstv, srcv,
                       mbuf, gidx, gbuf, sem, d)
        _sigmoid_rows(lo, a_hbm, acc, gbuf, d)
        pltpu.sync_copy(batch_hbm.at[pl.ds(lo, R)], bbuf)

        def prow(rg, _):
            bv = bbuf[pl.ds(rg * L, L)]
            for e in range(L):
                gid = bv[e]
                for f in range(d // L):
                    sl = pl.ds(f * L, L)
                    pool[gid, sl] = jnp.maximum(pool[gid, sl],
                                                acc[rg * L + e, sl])
            return 0

        lax.fori_loop(0, R // L, prow, 0)
        pltpu.sync_copy(pool, p_hbm.at[wid])

    return k(a2, b2, src, dst, batch_pad)


def kernel(x, edge_index, batch, W1, b1, W2, b2, Wo1, bo1, Wo2, bo2):
    src = edge_index[0]
    dst = edge_index[1]
    xp = jnp.pad(x, ((0, NPAD - N), (0, 5)))
    w1a = jnp.pad(W1[:3], ((0, 5), (0, 0)))
    w1b = jnp.pad(W1[3:], ((0, 5), (0, 0)))
    batch_pad = jnp.pad(batch, (0, NPAD - N), constant_values=G)

    a1, b1n = _tc_ab(xp, w1a, w1b, b1.reshape(1, -1), blk_rows=1280)
    h1 = _sc_layer1(a1, b1n, src, dst)
    a2, b2n = _tc_ab(h1, W2[:128], W2[128:], b2.reshape(1, -1),
                     blk_rows=1280)
    p = _sc_layer2(a2, b2n, src, dst, batch_pad)
    return _tc_post(p, Wo1, bo1.reshape(1, -1), Wo2, bo2.reshape(1, -1))
